# baseline (speedup 1.0000x reference)
import functools

import jax
import jax.numpy as jnp
from jax import lax
from jax.experimental import pallas as pl
from jax.experimental.pallas import tpu as pltpu
from jax.experimental.pallas import tpu_sc as plsc

_VOCAB = 1000000
_DIM = 64
_BATCH = 16384
_NEG = 5

_NSETS = 3 + _NEG
_TOTAL = _NSETS * _BATCH
_NC, _NS = 2, 16
_NW = _NC * _NS
_GCH = 128
_EPC = _GCH // _NSETS
_NCH = _TOTAL // _NW // _GCH
_EPW = _NCH * _EPC

_PREP_CH = 8192


def _prep_body(xt_ref, eye_ref, wmt_ref, out_ref):
    x = xt_ref[...].astype(jnp.bfloat16)
    e = lax.dot_general(x, eye_ref[...].astype(jnp.bfloat16),
                        (((0,), (0,)), ((), ())),
                        preferred_element_type=jnp.float32)
    m = lax.dot_general(x, wmt_ref[...].astype(jnp.bfloat16),
                        (((0,), (0,)), ((), ())),
                        preferred_element_type=jnp.float32)
    out_ref[...] = jnp.concatenate([e, m], axis=1)


def _prep(table_t, eye, wmt):
    nblk = (_VOCAB + _PREP_CH - 1) // _PREP_CH
    return pl.pallas_call(
        _prep_body,
        grid=(nblk,),
        in_specs=[
            pl.BlockSpec((_DIM, _PREP_CH), lambda i: (0, i)),
            pl.BlockSpec((_DIM, _DIM), lambda i: (0, 0)),
            pl.BlockSpec((_DIM, _DIM), lambda i: (0, 0)),
        ],
        out_specs=pl.BlockSpec((_PREP_CH, 2 * _DIM), lambda i: (i, 0)),
        out_shape=jax.ShapeDtypeStruct((_VOCAB, 2 * _DIM), jnp.float32),
    )(table_t, eye, wmt)


def _compute_chunk(buf, bvec, obuf):
    zero = jnp.zeros((16,), jnp.float32)
    for e in range(_EPC):
        r0 = e * _NSETS
        preds = []
        for c in range(4):
            p = (buf[r0, pl.ds(_DIM + 16 * c, 16)]
                 + buf[r0 + 1, pl.ds(_DIM + 16 * c, 16)]
                 + bvec[pl.ds(16 * c, 16)])
            preds.append(p)
        for d in range(6):
            src = r0 + 2 + d
            acc = preds[0] * buf[src, pl.ds(0, 16)]
            for c in range(1, 4):
                acc = acc + preds[c] * buf[src, pl.ds(16 * c, 16)]
            obuf[e, pl.ds(16 * d, 16)] = acc
        obuf[e, pl.ds(96, 16)] = zero
        obuf[e, pl.ds(112, 16)] = zero


def _sc_body(table2, idx_hbm, b_hbm, out_hbm,
             idx_v, buf_a, buf_b, bvec, obuf, sem_a, sem_b):
    wid = lax.axis_index("s") * _NC + lax.axis_index("c")
    pltpu.sync_copy(idx_hbm.at[pl.ds(wid * _NCH, _NCH)], idx_v)
    pltpu.sync_copy(b_hbm, bvec)

    def fire(j, buf, sem):
        pltpu.async_copy(table2.at[idx_v.at[j]], buf, sem)

    def drain(buf, sem):
        pltpu.make_async_copy(table2.at[pl.ds(0, _GCH)], buf, sem).wait()

    fire(0, buf_a, sem_a)
    fire(1, buf_b, sem_b)

    def body(j, carry):
        ja = 2 * j
        drain(buf_a, sem_a)
        _compute_chunk(buf_a, bvec, obuf)
        pltpu.sync_copy(obuf, out_hbm.at[pl.ds(wid * _EPW + ja * _EPC, _EPC)])

        @pl.when(ja + 2 < _NCH)
        def _():
            fire(ja + 2, buf_a, sem_a)

        drain(buf_b, sem_b)
        _compute_chunk(buf_b, bvec, obuf)
        pltpu.sync_copy(obuf, out_hbm.at[pl.ds(wid * _EPW + (ja + 1) * _EPC, _EPC)])

        @pl.when(ja + 3 < _NCH)
        def _():
            fire(ja + 3, buf_b, sem_b)

        return carry

    lax.fori_loop(0, _NCH // 2, body, 0)


@functools.cache
def _sc_gather_score():
    return pl.kernel(
        _sc_body,
        out_type=jax.ShapeDtypeStruct((_BATCH, 2 * _DIM), jnp.float32),
        mesh=plsc.VectorSubcoreMesh(core_axis_name="c", subcore_axis_name="s"),
        scratch_types=[
            pltpu.VMEM((_NCH, _GCH), jnp.int32),
            pltpu.VMEM((_GCH, 2 * _DIM), jnp.float32),
            pltpu.VMEM((_GCH, 2 * _DIM), jnp.float32),
            pltpu.VMEM((_DIM,), jnp.float32),
            pltpu.VMEM((_EPC, 2 * _DIM), jnp.float32),
            pltpu.SemaphoreType.DMA,
            pltpu.SemaphoreType.DMA,
        ],
    )


def _log_sigmoid(x):
    return jnp.minimum(x, 0.0) - jnp.log(1.0 + jnp.exp(-jnp.abs(x)))


_BS = 2048


def _finish_body(x_ref, out_ref):
    i = pl.program_id(0)

    @pl.when(i == 0)
    def _():
        out_ref[0, 0] = 0.0

    x = x_ref[...]
    group = lax.broadcasted_iota(jnp.int32, (_BS, 2 * _DIM), 1) // 16
    s = jnp.sum(jnp.where(group == 0, x, 0.0), axis=1)
    total = jnp.sum(_log_sigmoid(s))
    for d in range(1, 6):
        nd = jnp.sum(jnp.where(group == d, x, 0.0), axis=1)
        total = total + jnp.sum(_log_sigmoid(-nd))
    out_ref[0, 0] += total


def _finish(dots):
    return pl.pallas_call(
        _finish_body,
        grid=(_BATCH // _BS,),
        in_specs=[pl.BlockSpec((_BS, 2 * _DIM), lambda i: (i, 0))],
        out_specs=pl.BlockSpec((1, 1), lambda i: (0, 0), memory_space=pltpu.SMEM),
        out_shape=jax.ShapeDtypeStruct((1, 1), jnp.float32),
    )(dots)


def kernel(pos_u1, pos_u2, pos_v, neg_v, W_emb, W_map, b_map):
    idx_em = jnp.concatenate(
        [pos_u1[:, None], pos_u2[:, None], pos_v[:, None], neg_v],
        axis=1).astype(jnp.int32).reshape(_TOTAL // _GCH, _GCH)
    table_t = W_emb.T
    eye = jnp.eye(_DIM, dtype=jnp.float32)
    table2 = _prep(table_t, eye, W_map.T)
    dots = _sc_gather_score()(table2, idx_em, b_map)
    out = _finish(dots)
    return -out[0, 0]


# speedup vs baseline: 1.1175x; 1.1175x over previous
import functools

import jax
import jax.numpy as jnp
from jax import lax
from jax.experimental import pallas as pl
from jax.experimental.pallas import tpu as pltpu
from jax.experimental.pallas import tpu_sc as plsc

_VOCAB = 1000000
_DIM = 64
_BATCH = 16384
_NEG = 5

_NSETS = 3 + _NEG
_TOTAL = _NSETS * _BATCH
_NC, _NS = 2, 16
_NW = _NC * _NS
_GCH = 128
_EPC = _GCH // _NSETS
_NCH = _TOTAL // _NW // _GCH
_EPW = _NCH * _EPC

_PREP_CH = 8192


def _prep_body(xt_ref, ew_ref, out_ref):
    x = xt_ref[...].astype(jnp.bfloat16)
    out_ref[...] = lax.dot_general(x, ew_ref[...].astype(jnp.bfloat16),
                                   (((0,), (0,)), ((), ())),
                                   preferred_element_type=jnp.float32)


def _prep(table_t, ew):
    nblk = (_VOCAB + _PREP_CH - 1) // _PREP_CH
    return pl.pallas_call(
        _prep_body,
        grid=(nblk,),
        in_specs=[
            pl.BlockSpec((_DIM, _PREP_CH), lambda i: (0, i)),
            pl.BlockSpec((_DIM, 2 * _DIM), lambda i: (0, 0)),
        ],
        out_specs=pl.BlockSpec((_PREP_CH, 2 * _DIM), lambda i: (i, 0)),
        out_shape=jax.ShapeDtypeStruct((_VOCAB, 2 * _DIM), jnp.float32),
    )(table_t, ew)


def _compute_chunk(buf, bvec, obuf):
    zero = jnp.zeros((16,), jnp.float32)
    for e in range(_EPC):
        r0 = e * _NSETS
        preds = []
        for c in range(4):
            p = (buf[r0, pl.ds(_DIM + 16 * c, 16)]
                 + buf[r0 + 1, pl.ds(_DIM + 16 * c, 16)]
                 + bvec[pl.ds(16 * c, 16)])
            preds.append(p)
        for d in range(6):
            src = r0 + 2 + d
            acc = preds[0] * buf[src, pl.ds(0, 16)]
            for c in range(1, 4):
                acc = acc + preds[c] * buf[src, pl.ds(16 * c, 16)]
            obuf[e, pl.ds(16 * d, 16)] = acc
        obuf[e, pl.ds(96, 16)] = zero
        obuf[e, pl.ds(112, 16)] = zero


def _sc_body(table2, idx_hbm, b_hbm, out_hbm,
             idx_v, buf_a, buf_b, bvec, obuf, sem_a, sem_b):
    wid = lax.axis_index("s") * _NC + lax.axis_index("c")
    pltpu.sync_copy(idx_hbm.at[pl.ds(wid * _NCH, _NCH)], idx_v)
    pltpu.sync_copy(b_hbm, bvec)

    def fire(j, buf, sem):
        pltpu.async_copy(table2.at[idx_v.at[j]], buf, sem)

    def drain(buf, sem):
        pltpu.make_async_copy(table2.at[pl.ds(0, _GCH)], buf, sem).wait()

    fire(0, buf_a, sem_a)
    fire(1, buf_b, sem_b)

    def body(j, carry):
        ja = 2 * j
        drain(buf_a, sem_a)
        _compute_chunk(buf_a, bvec, obuf)
        pltpu.sync_copy(obuf, out_hbm.at[pl.ds(wid * _EPW + ja * _EPC, _EPC)])

        @pl.when(ja + 2 < _NCH)
        def _():
            fire(ja + 2, buf_a, sem_a)

        drain(buf_b, sem_b)
        _compute_chunk(buf_b, bvec, obuf)
        pltpu.sync_copy(obuf, out_hbm.at[pl.ds(wid * _EPW + (ja + 1) * _EPC, _EPC)])

        @pl.when(ja + 3 < _NCH)
        def _():
            fire(ja + 3, buf_b, sem_b)

        return carry

    lax.fori_loop(0, _NCH // 2, body, 0)


@functools.cache
def _sc_gather_score():
    return pl.kernel(
        _sc_body,
        out_type=jax.ShapeDtypeStruct((_BATCH, 2 * _DIM), jnp.float32),
        mesh=plsc.VectorSubcoreMesh(core_axis_name="c", subcore_axis_name="s"),
        scratch_types=[
            pltpu.VMEM((_NCH, _GCH), jnp.int32),
            pltpu.VMEM((_GCH, 2 * _DIM), jnp.float32),
            pltpu.VMEM((_GCH, 2 * _DIM), jnp.float32),
            pltpu.VMEM((_DIM,), jnp.float32),
            pltpu.VMEM((_EPC, 2 * _DIM), jnp.float32),
            pltpu.SemaphoreType.DMA,
            pltpu.SemaphoreType.DMA,
        ],
    )


def _log_sigmoid(x):
    return jnp.minimum(x, 0.0) - jnp.log(1.0 + jnp.exp(-jnp.abs(x)))


_BS = 2048


def _finish_body(x_ref, out_ref):
    i = pl.program_id(0)

    @pl.when(i == 0)
    def _():
        out_ref[0, 0] = 0.0

    x = x_ref[...]
    group = lax.broadcasted_iota(jnp.int32, (_BS, 2 * _DIM), 1) // 16
    s = jnp.sum(jnp.where(group == 0, x, 0.0), axis=1)
    total = jnp.sum(_log_sigmoid(s))
    for d in range(1, 6):
        nd = jnp.sum(jnp.where(group == d, x, 0.0), axis=1)
        total = total + jnp.sum(_log_sigmoid(-nd))
    out_ref[0, 0] += total


def _finish(dots):
    return pl.pallas_call(
        _finish_body,
        grid=(_BATCH // _BS,),
        in_specs=[pl.BlockSpec((_BS, 2 * _DIM), lambda i: (i, 0))],
        out_specs=pl.BlockSpec((1, 1), lambda i: (0, 0), memory_space=pltpu.SMEM),
        out_shape=jax.ShapeDtypeStruct((1, 1), jnp.float32),
    )(dots)


def kernel(pos_u1, pos_u2, pos_v, neg_v, W_emb, W_map, b_map):
    idx_em = jnp.concatenate(
        [pos_u1[:, None], pos_u2[:, None], pos_v[:, None], neg_v],
        axis=1).astype(jnp.int32).reshape(_TOTAL // _GCH, _GCH)
    table_t = W_emb.T
    ew = jnp.concatenate([jnp.eye(_DIM, dtype=jnp.float32), W_map.T], axis=1)
    table2 = _prep(table_t, ew)
    dots = _sc_gather_score()(table2, idx_em, b_map)
    out = _finish(dots)
    return -out[0, 0]


# speedup vs baseline: 1.2642x; 1.1313x over previous
import functools

import jax
import jax.numpy as jnp
from jax import lax
from jax.experimental import pallas as pl
from jax.experimental.pallas import tpu as pltpu
from jax.experimental.pallas import tpu_sc as plsc

_VOCAB = 1000000
_DIM = 64
_BATCH = 16384
_NEG = 5

_NSETS = 3 + _NEG
_TOTAL = _NSETS * _BATCH
_NC, _NS = 2, 16
_NW = _NC * _NS
_GCH = 128
_EPC = _GCH // _NSETS
_NCH = _TOTAL // _NW // _GCH
_EPW = _NCH * _EPC

_PREP_CH = 8192


def _prep_body(xt_ref, ew_ref, out_ref):
    x = xt_ref[...].astype(jnp.bfloat16)
    y = lax.dot_general(x, ew_ref[...].astype(jnp.bfloat16),
                        (((0,), (0,)), ((), ())),
                        preferred_element_type=jnp.float32)
    out_ref[...] = pltpu.bitcast(y.astype(jnp.bfloat16), jnp.int32)


def _prep(table_t, ew):
    nblk = (_VOCAB + _PREP_CH - 1) // _PREP_CH
    return pl.pallas_call(
        _prep_body,
        grid=(nblk,),
        in_specs=[
            pl.BlockSpec((_DIM, _PREP_CH), lambda i: (0, i)),
            pl.BlockSpec((_DIM, 2 * _DIM), lambda i: (0, 0)),
        ],
        out_specs=pl.BlockSpec((_PREP_CH // 2, 2 * _DIM), lambda i: (i, 0)),
        out_shape=jax.ShapeDtypeStruct((_VOCAB // 2, 2 * _DIM), jnp.int32),
    )(table_t, ew)


def _sc_body(table3, idxf_hbm, par_hbm, b_hbm, out_hbm,
             idxf_v, par_w, buf_a, buf_b, bvec, obuf, sem_a, sem_b):
    wid = lax.axis_index("s") * _NC + lax.axis_index("c")
    pltpu.sync_copy(idxf_hbm.at[pl.ds(wid * _NCH, _NCH)], idxf_v)
    pltpu.sync_copy(par_hbm.at[pl.ds(wid * _EPW, _EPW)], par_w)
    pltpu.sync_copy(b_hbm, bvec)

    def fire(j, buf, sem):
        pltpu.async_copy(table3.at[idxf_v.at[j]], buf, sem)

    def drain(buf, sem):
        pltpu.make_async_copy(table3.at[pl.ds(0, _GCH)], buf, sem).wait()

    def compute_chunk(j, buf):
        def elem(e, carry):
            r0 = e * _NSETS

            def pvec(k):
                return par_w[j * _EPC + e, pl.ds(16 * k, 16)] != 0

            def half(k, off, odd):
                w = buf[r0 + k, pl.ds(off, 16)]
                hi = lax.bitcast_convert_type(
                    jnp.bitwise_and(w, jnp.int32(-65536)), jnp.float32)
                lo = lax.bitcast_convert_type(
                    lax.shift_left(w, jnp.int32(16)), jnp.float32)
                return jnp.where(odd, hi, lo)

            p0 = pvec(0)
            p1 = pvec(1)
            preds = []
            for c in range(4):
                p = (half(0, _DIM + 16 * c, p0)
                     + half(1, _DIM + 16 * c, p1)
                     + bvec[pl.ds(16 * c, 16)])
                preds.append(p)
            for d in range(6):
                k = 2 + d
                pk = pvec(k)
                acc = preds[0] * half(k, 0, pk)
                for c in range(1, 4):
                    acc = acc + preds[c] * half(k, 16 * c, pk)
                obuf[e, pl.ds(16 * d, 16)] = acc
            zero = jnp.zeros((16,), jnp.float32)
            obuf[e, pl.ds(96, 16)] = zero
            obuf[e, pl.ds(112, 16)] = zero
            return carry

        lax.fori_loop(0, _EPC, elem, 0)

    fire(0, buf_a, sem_a)
    fire(1, buf_b, sem_b)

    def body(j, carry):
        ja = 2 * j
        drain(buf_a, sem_a)
        compute_chunk(ja, buf_a)
        pltpu.sync_copy(obuf, out_hbm.at[pl.ds(wid * _EPW + ja * _EPC, _EPC)])

        @pl.when(ja + 2 < _NCH)
        def _():
            fire(ja + 2, buf_a, sem_a)

        drain(buf_b, sem_b)
        compute_chunk(ja + 1, buf_b)
        pltpu.sync_copy(obuf, out_hbm.at[pl.ds(wid * _EPW + (ja + 1) * _EPC, _EPC)])

        @pl.when(ja + 3 < _NCH)
        def _():
            fire(ja + 3, buf_b, sem_b)

        return carry

    lax.fori_loop(0, _NCH // 2, body, 0)


@functools.cache
def _sc_gather_score():
    return pl.kernel(
        _sc_body,
        out_type=jax.ShapeDtypeStruct((_BATCH, 2 * _DIM), jnp.float32),
        mesh=plsc.VectorSubcoreMesh(core_axis_name="c", subcore_axis_name="s"),
        scratch_types=[
            pltpu.VMEM((_NCH, _GCH), jnp.int32),
            pltpu.VMEM((_EPW, 2 * _DIM), jnp.int32),
            pltpu.VMEM((_GCH, 2 * _DIM), jnp.int32),
            pltpu.VMEM((_GCH, 2 * _DIM), jnp.int32),
            pltpu.VMEM((_DIM,), jnp.float32),
            pltpu.VMEM((_EPC, 2 * _DIM), jnp.float32),
            pltpu.SemaphoreType.DMA,
            pltpu.SemaphoreType.DMA,
        ],
    )


def _log_sigmoid(x):
    return jnp.minimum(x, 0.0) - jnp.log(1.0 + jnp.exp(-jnp.abs(x)))


_BS = 2048


def _finish_body(x_ref, out_ref):
    i = pl.program_id(0)

    @pl.when(i == 0)
    def _():
        out_ref[0, 0] = 0.0

    x = x_ref[...]
    group = lax.broadcasted_iota(jnp.int32, (_BS, 2 * _DIM), 1) // 16
    s = jnp.sum(jnp.where(group == 0, x, 0.0), axis=1)
    total = jnp.sum(_log_sigmoid(s))
    for d in range(1, 6):
        nd = jnp.sum(jnp.where(group == d, x, 0.0), axis=1)
        total = total + jnp.sum(_log_sigmoid(-nd))
    out_ref[0, 0] += total


def _finish(dots):
    return pl.pallas_call(
        _finish_body,
        grid=(_BATCH // _BS,),
        in_specs=[pl.BlockSpec((_BS, 2 * _DIM), lambda i: (i, 0))],
        out_specs=pl.BlockSpec((1, 1), lambda i: (0, 0), memory_space=pltpu.SMEM),
        out_shape=jax.ShapeDtypeStruct((1, 1), jnp.float32),
    )(dots)


def kernel(pos_u1, pos_u2, pos_v, neg_v, W_emb, W_map, b_map):
    idx_em = jnp.concatenate(
        [pos_u1[:, None], pos_u2[:, None], pos_v[:, None], neg_v],
        axis=1).astype(jnp.int32).reshape(-1)
    idx_f = (idx_em >> 1).reshape(_TOTAL // _GCH, _GCH)
    par = jnp.repeat((idx_em & 1).reshape(_BATCH, _NSETS), 16, axis=1)
    table_t = W_emb.T
    ew = jnp.concatenate([jnp.eye(_DIM, dtype=jnp.float32), W_map.T], axis=1)
    table3 = _prep(table_t, ew)
    dots = _sc_gather_score()(table3, idx_f, par, b_map)
    out = _finish(dots)
    return -out[0, 0]


# speedup vs baseline: 1.4286x; 1.1301x over previous
import functools

import jax
import jax.numpy as jnp
from jax import lax
from jax.experimental import pallas as pl
from jax.experimental.pallas import tpu as pltpu
from jax.experimental.pallas import tpu_sc as plsc

_VOCAB = 1000000
_DIM = 64
_BATCH = 16384
_NEG = 5

_NSETS = 3 + _NEG
_TOTAL = _NSETS * _BATCH
_NC, _NS = 2, 16
_NW = _NC * _NS
_GCH = 128
_EPC = _GCH // _NSETS
_NCH = _TOTAL // _NW // _GCH
_EPW = _NCH * _EPC

_PREP_CH = 16384


def _prep_body(xt_ref, ew_ref, out_ref):
    x = xt_ref[...].astype(jnp.bfloat16)
    y = lax.dot_general(x, ew_ref[...].astype(jnp.bfloat16),
                        (((0,), (0,)), ((), ())),
                        preferred_element_type=jnp.float32)
    out_ref[...] = pltpu.bitcast(y.astype(jnp.bfloat16), jnp.int32)


def _prep(table_t, ew):
    nblk = (_VOCAB + _PREP_CH - 1) // _PREP_CH
    return pl.pallas_call(
        _prep_body,
        grid=(nblk,),
        in_specs=[
            pl.BlockSpec((_DIM, _PREP_CH), lambda i: (0, i)),
            pl.BlockSpec((_DIM, 2 * _DIM), lambda i: (0, 0)),
        ],
        out_specs=pl.BlockSpec((_PREP_CH // 2, 2 * _DIM), lambda i: (i, 0)),
        out_shape=jax.ShapeDtypeStruct((_VOCAB // 2, 2 * _DIM), jnp.int32),
    )(table_t, ew)


def _sc_body(table3, idxf_hbm, par_hbm, b_hbm, out_hbm,
             idxf_v, par_w, buf_a, buf_b, bvec, obuf, sem_a, sem_b):
    wid = lax.axis_index("s") * _NC + lax.axis_index("c")
    pltpu.sync_copy(idxf_hbm.at[pl.ds(wid * _NCH, _NCH)], idxf_v)
    pltpu.sync_copy(par_hbm.at[pl.ds(wid * _EPW, _EPW)], par_w)
    pltpu.sync_copy(b_hbm, bvec)

    def fire(j, buf, sem):
        pltpu.async_copy(table3.at[idxf_v.at[j]], buf, sem)

    def drain(buf, sem):
        pltpu.make_async_copy(table3.at[pl.ds(0, _GCH)], buf, sem).wait()

    def compute_chunk(j, buf):
        def elem(e, carry):
            r0 = e * _NSETS

            def pvec(k):
                return par_w[j * _EPC + e, pl.ds(16 * k, 16)] != 0

            def half(k, off, odd):
                w = buf[r0 + k, pl.ds(off, 16)]
                hi = lax.bitcast_convert_type(
                    jnp.bitwise_and(w, jnp.int32(-65536)), jnp.float32)
                lo = lax.bitcast_convert_type(
                    lax.shift_left(w, jnp.int32(16)), jnp.float32)
                return jnp.where(odd, hi, lo)

            p0 = pvec(0)
            p1 = pvec(1)
            preds = []
            for c in range(4):
                p = (half(0, _DIM + 16 * c, p0)
                     + half(1, _DIM + 16 * c, p1)
                     + bvec[pl.ds(16 * c, 16)])
                preds.append(p)
            for d in range(6):
                k = 2 + d
                pk = pvec(k)
                acc = preds[0] * half(k, 0, pk)
                for c in range(1, 4):
                    acc = acc + preds[c] * half(k, 16 * c, pk)
                obuf[e, pl.ds(16 * d, 16)] = acc
            zero = jnp.zeros((16,), jnp.float32)
            obuf[e, pl.ds(96, 16)] = zero
            obuf[e, pl.ds(112, 16)] = zero
            return carry

        lax.fori_loop(0, _EPC, elem, 0)

    fire(0, buf_a, sem_a)
    fire(1, buf_b, sem_b)

    def body(j, carry):
        ja = 2 * j
        drain(buf_a, sem_a)
        compute_chunk(ja, buf_a)
        pltpu.sync_copy(obuf, out_hbm.at[pl.ds(wid * _EPW + ja * _EPC, _EPC)])

        @pl.when(ja + 2 < _NCH)
        def _():
            fire(ja + 2, buf_a, sem_a)

        drain(buf_b, sem_b)
        compute_chunk(ja + 1, buf_b)
        pltpu.sync_copy(obuf, out_hbm.at[pl.ds(wid * _EPW + (ja + 1) * _EPC, _EPC)])

        @pl.when(ja + 3 < _NCH)
        def _():
            fire(ja + 3, buf_b, sem_b)

        return carry

    lax.fori_loop(0, _NCH // 2, body, 0)


@functools.cache
def _sc_gather_score():
    return pl.kernel(
        _sc_body,
        out_type=jax.ShapeDtypeStruct((_BATCH, 2 * _DIM), jnp.float32),
        mesh=plsc.VectorSubcoreMesh(core_axis_name="c", subcore_axis_name="s"),
        scratch_types=[
            pltpu.VMEM((_NCH, _GCH), jnp.int32),
            pltpu.VMEM((_EPW, 2 * _DIM), jnp.int32),
            pltpu.VMEM((_GCH, 2 * _DIM), jnp.int32),
            pltpu.VMEM((_GCH, 2 * _DIM), jnp.int32),
            pltpu.VMEM((_DIM,), jnp.float32),
            pltpu.VMEM((_EPC, 2 * _DIM), jnp.float32),
            pltpu.SemaphoreType.DMA,
            pltpu.SemaphoreType.DMA,
        ],
    )


def _log_sigmoid(x):
    return jnp.minimum(x, 0.0) - jnp.log(1.0 + jnp.exp(-jnp.abs(x)))


_BS = 2048


def _finish_body(x_ref, out_ref):
    i = pl.program_id(0)

    @pl.when(i == 0)
    def _():
        out_ref[0, 0] = 0.0

    x = x_ref[...]
    group = lax.broadcasted_iota(jnp.int32, (_BS, 2 * _DIM), 1) // 16
    s = jnp.sum(jnp.where(group == 0, x, 0.0), axis=1)
    total = jnp.sum(_log_sigmoid(s))
    for d in range(1, 6):
        nd = jnp.sum(jnp.where(group == d, x, 0.0), axis=1)
        total = total + jnp.sum(_log_sigmoid(-nd))
    out_ref[0, 0] += total


def _finish(dots):
    return pl.pallas_call(
        _finish_body,
        grid=(_BATCH // _BS,),
        in_specs=[pl.BlockSpec((_BS, 2 * _DIM), lambda i: (i, 0))],
        out_specs=pl.BlockSpec((1, 1), lambda i: (0, 0), memory_space=pltpu.SMEM),
        out_shape=jax.ShapeDtypeStruct((1, 1), jnp.float32),
    )(dots)


def kernel(pos_u1, pos_u2, pos_v, neg_v, W_emb, W_map, b_map):
    idx_em = jnp.concatenate(
        [pos_u1[:, None], pos_u2[:, None], pos_v[:, None], neg_v],
        axis=1).astype(jnp.int32).reshape(-1)
    idx_f = (idx_em >> 1).reshape(_TOTAL // _GCH, _GCH)
    par = jnp.repeat((idx_em & 1).reshape(_BATCH, _NSETS), 16, axis=1)
    table_t = W_emb.T
    ew = jnp.concatenate([jnp.eye(_DIM, dtype=jnp.float32), W_map.T], axis=1)
    table3 = _prep(table_t, ew)
    dots = _sc_gather_score()(table3, idx_f, par, b_map)
    out = _finish(dots)
    return -out[0, 0]


# speedup vs baseline: 1.4582x; 1.0207x over previous
import functools

import jax
import jax.numpy as jnp
from jax import lax
from jax.experimental import pallas as pl
from jax.experimental.pallas import tpu as pltpu
from jax.experimental.pallas import tpu_sc as plsc

_VOCAB = 1000000
_DIM = 64
_BATCH = 16384
_NEG = 5

_NSETS = 3 + _NEG
_TOTAL = _NSETS * _BATCH
_NC, _NS = 2, 16
_NW = _NC * _NS
_GCH = 128
_EPC = _GCH // _NSETS
_NCH = _TOTAL // _NW // _GCH
_EPW = _NCH * _EPC

_PREP_CH = 32768


def _prep_body(xt_ref, ew_ref, out_ref):
    x = xt_ref[...].astype(jnp.bfloat16)
    y = lax.dot_general(x, ew_ref[...].astype(jnp.bfloat16),
                        (((0,), (0,)), ((), ())),
                        preferred_element_type=jnp.float32)
    out_ref[...] = pltpu.bitcast(y.astype(jnp.bfloat16), jnp.int32)


def _prep(table_t, ew):
    nblk = (_VOCAB + _PREP_CH - 1) // _PREP_CH
    return pl.pallas_call(
        _prep_body,
        grid=(nblk,),
        in_specs=[
            pl.BlockSpec((_DIM, _PREP_CH), lambda i: (0, i)),
            pl.BlockSpec((_DIM, 2 * _DIM), lambda i: (0, 0)),
        ],
        out_specs=pl.BlockSpec((_PREP_CH // 2, 2 * _DIM), lambda i: (i, 0)),
        out_shape=jax.ShapeDtypeStruct((_VOCAB // 2, 2 * _DIM), jnp.int32),
    )(table_t, ew)


def _sc_body(table3, idxf_hbm, par_hbm, b_hbm, out_hbm,
             idxf_v, par_w, buf_a, buf_b, bvec, obuf, sem_a, sem_b):
    wid = lax.axis_index("s") * _NC + lax.axis_index("c")
    pltpu.sync_copy(idxf_hbm.at[pl.ds(wid * _NCH, _NCH)], idxf_v)
    pltpu.sync_copy(par_hbm.at[pl.ds(wid * _EPW, _EPW)], par_w)
    pltpu.sync_copy(b_hbm, bvec)

    def fire(j, buf, sem):
        pltpu.async_copy(table3.at[idxf_v.at[j]], buf, sem)

    def drain(buf, sem):
        pltpu.make_async_copy(table3.at[pl.ds(0, _GCH)], buf, sem).wait()

    def compute_chunk(j, buf):
        def elem(e, carry):
            r0 = e * _NSETS

            def pvec(k):
                return par_w[j * _EPC + e, pl.ds(16 * k, 16)] != 0

            def half(k, off, odd):
                w = buf[r0 + k, pl.ds(off, 16)]
                hi = lax.bitcast_convert_type(
                    jnp.bitwise_and(w, jnp.int32(-65536)), jnp.float32)
                lo = lax.bitcast_convert_type(
                    lax.shift_left(w, jnp.int32(16)), jnp.float32)
                return jnp.where(odd, hi, lo)

            p0 = pvec(0)
            p1 = pvec(1)
            preds = []
            for c in range(4):
                p = (half(0, _DIM + 16 * c, p0)
                     + half(1, _DIM + 16 * c, p1)
                     + bvec[pl.ds(16 * c, 16)])
                preds.append(p)
            for d in range(6):
                k = 2 + d
                pk = pvec(k)
                acc = preds[0] * half(k, 0, pk)
                for c in range(1, 4):
                    acc = acc + preds[c] * half(k, 16 * c, pk)
                obuf[e, pl.ds(16 * d, 16)] = acc
            zero = jnp.zeros((16,), jnp.float32)
            obuf[e, pl.ds(96, 16)] = zero
            obuf[e, pl.ds(112, 16)] = zero
            return carry

        lax.fori_loop(0, _EPC, elem, 0)

    fire(0, buf_a, sem_a)
    fire(1, buf_b, sem_b)

    def body(j, carry):
        ja = 2 * j
        drain(buf_a, sem_a)
        compute_chunk(ja, buf_a)
        pltpu.sync_copy(obuf, out_hbm.at[pl.ds(wid * _EPW + ja * _EPC, _EPC)])

        @pl.when(ja + 2 < _NCH)
        def _():
            fire(ja + 2, buf_a, sem_a)

        drain(buf_b, sem_b)
        compute_chunk(ja + 1, buf_b)
        pltpu.sync_copy(obuf, out_hbm.at[pl.ds(wid * _EPW + (ja + 1) * _EPC, _EPC)])

        @pl.when(ja + 3 < _NCH)
        def _():
            fire(ja + 3, buf_b, sem_b)

        return carry

    lax.fori_loop(0, _NCH // 2, body, 0)


@functools.cache
def _sc_gather_score():
    return pl.kernel(
        _sc_body,
        out_type=jax.ShapeDtypeStruct((_BATCH, 2 * _DIM), jnp.float32),
        mesh=plsc.VectorSubcoreMesh(core_axis_name="c", subcore_axis_name="s"),
        scratch_types=[
            pltpu.VMEM((_NCH, _GCH), jnp.int32),
            pltpu.VMEM((_EPW, 2 * _DIM), jnp.int32),
            pltpu.VMEM((_GCH, 2 * _DIM), jnp.int32),
            pltpu.VMEM((_GCH, 2 * _DIM), jnp.int32),
            pltpu.VMEM((_DIM,), jnp.float32),
            pltpu.VMEM((_EPC, 2 * _DIM), jnp.float32),
            pltpu.SemaphoreType.DMA,
            pltpu.SemaphoreType.DMA,
        ],
    )


def _log_sigmoid(x):
    return jnp.minimum(x, 0.0) - jnp.log(1.0 + jnp.exp(-jnp.abs(x)))


_BS = 2048


def _finish_body(x_ref, out_ref):
    i = pl.program_id(0)

    @pl.when(i == 0)
    def _():
        out_ref[0, 0] = 0.0

    x = x_ref[...]
    group = lax.broadcasted_iota(jnp.int32, (_BS, 2 * _DIM), 1) // 16
    s = jnp.sum(jnp.where(group == 0, x, 0.0), axis=1)
    total = jnp.sum(_log_sigmoid(s))
    for d in range(1, 6):
        nd = jnp.sum(jnp.where(group == d, x, 0.0), axis=1)
        total = total + jnp.sum(_log_sigmoid(-nd))
    out_ref[0, 0] += total


def _finish(dots):
    return pl.pallas_call(
        _finish_body,
        grid=(_BATCH // _BS,),
        in_specs=[pl.BlockSpec((_BS, 2 * _DIM), lambda i: (i, 0))],
        out_specs=pl.BlockSpec((1, 1), lambda i: (0, 0), memory_space=pltpu.SMEM),
        out_shape=jax.ShapeDtypeStruct((1, 1), jnp.float32),
    )(dots)


def kernel(pos_u1, pos_u2, pos_v, neg_v, W_emb, W_map, b_map):
    idx_em = jnp.concatenate(
        [pos_u1[:, None], pos_u2[:, None], pos_v[:, None], neg_v],
        axis=1).astype(jnp.int32).reshape(-1)
    idx_f = (idx_em >> 1).reshape(_TOTAL // _GCH, _GCH)
    par = jnp.repeat((idx_em & 1).reshape(_BATCH, _NSETS), 16, axis=1)
    table_t = W_emb.T
    ew = jnp.concatenate([jnp.eye(_DIM, dtype=jnp.float32), W_map.T], axis=1)
    table3 = _prep(table_t, ew)
    dots = _sc_gather_score()(table3, idx_f, par, b_map)
    out = _finish(dots)
    return -out[0, 0]
